# baseline (speedup 1.0000x reference)
import jax
import jax.numpy as jnp
from jax.experimental import pallas as pl
from jax.experimental.pallas import tpu as pltpu

_VMEM_LIMIT = 64 * 1024 * 1024
_TH = 8


def _make_fused_kernel(H, W, n):
    W2 = 2 * W
    HW = H * W
    NB = H // _TH

    def _one_image(xi, s3_ref, wbd_ref, w2bd_ref, r_ref):
        lane = jax.lax.broadcasted_iota(jnp.int32, (1, W2), 1)
        even = (lane % 2) == 0
        col = jnp.sum(xi, axis=0, keepdims=True)
        sqcol = jnp.sum(xi * xi, axis=0, keepdims=True)
        m0 = jnp.sum(jnp.where(even, col, 0.0)) * (1.0 / HW)
        m1 = jnp.sum(jnp.where(even, 0.0, col)) * (1.0 / HW)
        mean_lane = jnp.where(even, m0, m1)
        e0 = jnp.sum(jnp.where(even, sqcol, 0.0))
        e1 = jnp.sum(jnp.where(even, 0.0, sqcol))
        v0 = (e0 - HW * m0 * m0) * (1.0 / (HW - 1))
        v1 = (e1 - HW * m1 * m1) * (1.0 / (HW - 1))
        istd_lane = jnp.where(even, jax.lax.rsqrt(v0), jax.lax.rsqrt(v1))
        xn = (xi - mean_lane) * istd_lane
        planes = jnp.dot(xn, s3_ref[...], preferred_element_type=jnp.float32)
        G = min(8, NB)
        WG = G * W
        ones_row = jnp.ones((1, WG), jnp.float32)
        ones_bf = jnp.ones((1, WG), jnp.bfloat16)
        def rslice(r, k):
            lo, hi = k * W, (k + 1) * W
            if r < 0:
                return jnp.concatenate(
                    [jnp.zeros((-r, W), jnp.float32), planes[0:r + _TH, lo:hi]],
                    axis=0)
            if r + _TH > H:
                return jnp.concatenate(
                    [planes[r:H, lo:hi], jnp.zeros((r + _TH - H, W), jnp.float32)],
                    axis=0)
            return planes[r:r + _TH, lo:hi]

        y0s = []
        y1s = []
        for t in range(NB // G):
            r0 = t * G * _TH
            pieces = [
                jnp.concatenate(
                    [rslice(r0 + g * _TH + dh - 1, k) for g in range(G)], axis=1)
                for k in range(6) for dh in range(3)
            ]
            pieces.append(ones_row)
            p_s = jnp.concatenate(pieces, axis=0)
            h1 = jnp.dot(wbd_ref[...], p_s, preferred_element_type=jnp.float32)
            hb = h1.astype(jnp.bfloat16)
            sb = hb * jax.nn.sigmoid(hb)
            h2 = jnp.concatenate([sb, ones_bf], axis=0)
            y2 = jnp.dot(w2bd_ref[...], h2, preferred_element_type=jnp.float32)
            for g in range(G):
                y0s.append(y2[0:_TH, g * W:(g + 1) * W])
                y1s.append(y2[_TH:2 * _TH, g * W:(g + 1) * W])
        y0 = jnp.concatenate(y0s, axis=0)
        y1 = jnp.concatenate(y1s, axis=0)
        zcat = jnp.concatenate([y0, y1], axis=1).astype(jnp.bfloat16)
        out = jnp.dot(zcat, r_ref[...], preferred_element_type=jnp.float32)
        std_lane = jnp.where(even, jnp.sqrt(v0), jnp.sqrt(v1))
        return out * std_lane + mean_lane

    def _body(x_ref, s3_ref, wbd_ref, w2bd_ref, r_ref, o_ref):
        for i in range(x_ref.shape[0]):
            o_ref[i] = _one_image(x_ref[i], s3_ref, wbd_ref, w2bd_ref, r_ref)

    return _body


def kernel(x, w1, b1, w2, b2):
    b, c, H, W, two = x.shape
    n = w1.shape[0]
    TH = _TH
    eye = jnp.eye(TH, dtype=jnp.float32)
    w1t = w1.reshape(n, 2, 3, 3)
    wbd = jnp.einsum("ncxw,hk->hncwxk", w1t, eye).reshape(TH * n, 144)
    bias1 = jnp.tile(b1.reshape(n), (TH,)).reshape(TH * n, 1)
    wbd = jnp.concatenate([wbd, bias1], axis=1)
    w2bd = jnp.einsum("cn,hk->chkn", w2, eye).reshape(2 * TH, TH * n)
    bias2 = jnp.repeat(b2.reshape(2), TH).reshape(2 * TH, 1)
    w2bd = jnp.concatenate([w2bd, bias2], axis=1).astype(jnp.bfloat16)
    jj = jnp.arange(6 * W)
    c_of = jj // (3 * W)
    dw_of = (jj // W) % 3
    w_of = jj % W
    wsrc = w_of + dw_of - 1
    src = 2 * wsrc + c_of
    valid = (wsrc >= 0) & (wsrc < W)
    s3 = ((jnp.arange(2 * W)[:, None] == src[None, :]) & valid[None, :]
          ).astype(jnp.float32)
    jo = jnp.arange(2 * W)
    rsrc = (jo % 2) * W + jo // 2
    rmat = (jnp.arange(2 * W)[:, None] == rsrc[None, :]).astype(jnp.bfloat16)

    xi = x.reshape(b, H, 2 * W)
    body = _make_fused_kernel(H, W, n)
    BB = 4 if b % 4 == 0 else 1

    def _run(xpart):
        nb = xpart.shape[0]
        return pl.pallas_call(
            body,
            out_shape=jax.ShapeDtypeStruct((nb, H, 2 * W), jnp.float32),
            grid=(nb // BB,),
            in_specs=[
                pl.BlockSpec((BB, H, 2 * W), lambda i: (i, 0, 0)),
                pl.BlockSpec((2 * W, 6 * W), lambda i: (0, 0)),
                pl.BlockSpec((TH * n, 145), lambda i: (0, 0)),
                pl.BlockSpec((2 * TH, TH * n + 1), lambda i: (0, 0)),
                pl.BlockSpec((2 * W, 2 * W), lambda i: (0, 0)),
            ],
            out_specs=pl.BlockSpec((BB, H, 2 * W), lambda i: (i, 0, 0)),
            compiler_params=pltpu.CompilerParams(
                dimension_semantics=("parallel",),
                vmem_limit_bytes=_VMEM_LIMIT),
        )(xpart, s3, wbd, w2bd, rmat)

    out = _run(xi)
    return out.reshape(b, c, H, W, 2)


# speedup vs baseline: 1.0705x; 1.0705x over previous
import jax
import jax.numpy as jnp
from jax.experimental import pallas as pl
from jax.experimental.pallas import tpu as pltpu

_VMEM_LIMIT = 64 * 1024 * 1024
_TH = 8


def _make_fused_kernel(H, W, n):
    W2 = 2 * W
    HW = H * W
    NB = H // _TH

    def _one_image(xi, s3_ref, wbd_ref, w2bd_ref, r_ref):
        lane = jax.lax.broadcasted_iota(jnp.int32, (1, W2), 1)
        even = (lane % 2) == 0
        col = jnp.sum(xi, axis=0, keepdims=True)
        sqcol = jnp.sum(xi * xi, axis=0, keepdims=True)
        m0 = jnp.sum(jnp.where(even, col, 0.0)) * (1.0 / HW)
        m1 = jnp.sum(jnp.where(even, 0.0, col)) * (1.0 / HW)
        mean_lane = jnp.where(even, m0, m1)
        e0 = jnp.sum(jnp.where(even, sqcol, 0.0))
        e1 = jnp.sum(jnp.where(even, 0.0, sqcol))
        v0 = (e0 - HW * m0 * m0) * (1.0 / (HW - 1))
        v1 = (e1 - HW * m1 * m1) * (1.0 / (HW - 1))
        istd_lane = jnp.where(even, jax.lax.rsqrt(v0), jax.lax.rsqrt(v1))
        xn = ((xi - mean_lane) * istd_lane).astype(jnp.bfloat16)
        planes = jnp.dot(xn, s3_ref[...], preferred_element_type=jnp.float32)
        G = min(16, NB)
        WG = G * W
        ones_row = jnp.ones((1, WG), jnp.float32)
        ones_bf = jnp.ones((1, WG), jnp.bfloat16)
        def rslice(r, k):
            lo, hi = k * W, (k + 1) * W
            if r < 0:
                return jnp.concatenate(
                    [jnp.zeros((-r, W), jnp.float32), planes[0:r + _TH, lo:hi]],
                    axis=0)
            if r + _TH > H:
                return jnp.concatenate(
                    [planes[r:H, lo:hi], jnp.zeros((r + _TH - H, W), jnp.float32)],
                    axis=0)
            return planes[r:r + _TH, lo:hi]

        y0s = []
        y1s = []
        for t in range(NB // G):
            r0 = t * G * _TH
            pieces = [
                jnp.concatenate(
                    [rslice(r0 + g * _TH + dh - 1, k) for g in range(G)], axis=1)
                for k in range(6) for dh in range(3)
            ]
            pieces.append(ones_row)
            p_s = jnp.concatenate(pieces, axis=0).astype(jnp.bfloat16)
            h1 = jnp.dot(wbd_ref[...], p_s, preferred_element_type=jnp.float32)
            hb = h1.astype(jnp.bfloat16)
            sb = hb * jax.nn.sigmoid(hb)
            h2 = jnp.concatenate([sb, ones_bf], axis=0)
            y2 = jnp.dot(w2bd_ref[...], h2, preferred_element_type=jnp.float32)
            for g in range(G):
                y0s.append(y2[0:_TH, g * W:(g + 1) * W])
                y1s.append(y2[_TH:2 * _TH, g * W:(g + 1) * W])
        y0 = jnp.concatenate(y0s, axis=0)
        y1 = jnp.concatenate(y1s, axis=0)
        zcat = jnp.concatenate([y0, y1], axis=1).astype(jnp.bfloat16)
        out = jnp.dot(zcat, r_ref[...], preferred_element_type=jnp.float32)
        std_lane = jnp.where(even, jnp.sqrt(v0), jnp.sqrt(v1))
        return out * std_lane + mean_lane

    def _body(x_ref, s3_ref, wbd_ref, w2bd_ref, r_ref, o_ref):
        for i in range(x_ref.shape[0]):
            o_ref[i] = _one_image(x_ref[i], s3_ref, wbd_ref, w2bd_ref, r_ref)

    return _body


def kernel(x, w1, b1, w2, b2):
    b, c, H, W, two = x.shape
    n = w1.shape[0]
    TH = _TH
    eye = jnp.eye(TH, dtype=jnp.float32)
    w1t = w1.reshape(n, 2, 3, 3)
    wbd = jnp.einsum("ncxw,hk->hncwxk", w1t, eye).reshape(TH * n, 144)
    bias1 = jnp.tile(b1.reshape(n), (TH,)).reshape(TH * n, 1)
    wbd = jnp.concatenate([wbd, bias1], axis=1).astype(jnp.bfloat16)
    w2bd = jnp.einsum("cn,hk->chkn", w2, eye).reshape(2 * TH, TH * n)
    bias2 = jnp.repeat(b2.reshape(2), TH).reshape(2 * TH, 1)
    w2bd = jnp.concatenate([w2bd, bias2], axis=1).astype(jnp.bfloat16)
    jj = jnp.arange(6 * W)
    c_of = jj // (3 * W)
    dw_of = (jj // W) % 3
    w_of = jj % W
    wsrc = w_of + dw_of - 1
    src = 2 * wsrc + c_of
    valid = (wsrc >= 0) & (wsrc < W)
    s3 = ((jnp.arange(2 * W)[:, None] == src[None, :]) & valid[None, :]
          ).astype(jnp.bfloat16)
    jo = jnp.arange(2 * W)
    rsrc = (jo % 2) * W + jo // 2
    rmat = (jnp.arange(2 * W)[:, None] == rsrc[None, :]).astype(jnp.bfloat16)

    xi = x.reshape(b, H, 2 * W)
    body = _make_fused_kernel(H, W, n)
    BB = 4 if b % 4 == 0 else 1

    def _run(xpart):
        nb = xpart.shape[0]
        return pl.pallas_call(
            body,
            out_shape=jax.ShapeDtypeStruct((nb, H, 2 * W), jnp.float32),
            grid=(nb // BB,),
            in_specs=[
                pl.BlockSpec((BB, H, 2 * W), lambda i: (i, 0, 0)),
                pl.BlockSpec((2 * W, 6 * W), lambda i: (0, 0)),
                pl.BlockSpec((TH * n, 145), lambda i: (0, 0)),
                pl.BlockSpec((2 * TH, TH * n + 1), lambda i: (0, 0)),
                pl.BlockSpec((2 * W, 2 * W), lambda i: (0, 0)),
            ],
            out_specs=pl.BlockSpec((BB, H, 2 * W), lambda i: (i, 0, 0)),
            compiler_params=pltpu.CompilerParams(
                dimension_semantics=("parallel",),
                vmem_limit_bytes=_VMEM_LIMIT),
        )(xpart, s3, wbd, w2bd, rmat)

    out = _run(xi)
    return out.reshape(b, c, H, W, 2)


# speedup vs baseline: 1.0840x; 1.0127x over previous
import jax
import jax.numpy as jnp
from jax.experimental import pallas as pl
from jax.experimental.pallas import tpu as pltpu

_VMEM_LIMIT = 64 * 1024 * 1024
_TH = 8


def _make_fused_kernel(H, W, n):
    W2 = 2 * W
    HW = H * W
    NB = H // _TH

    def _one_image(xi, s3_ref, wbd_ref, w2bd_ref, r_ref):
        lane = jax.lax.broadcasted_iota(jnp.int32, (1, W2), 1)
        even = (lane % 2) == 0
        col = jnp.sum(xi, axis=0, keepdims=True)
        sqcol = jnp.sum(xi * xi, axis=0, keepdims=True)
        m0 = jnp.sum(jnp.where(even, col, 0.0)) * (1.0 / HW)
        m1 = jnp.sum(jnp.where(even, 0.0, col)) * (1.0 / HW)
        mean_lane = jnp.where(even, m0, m1)
        e0 = jnp.sum(jnp.where(even, sqcol, 0.0))
        e1 = jnp.sum(jnp.where(even, 0.0, sqcol))
        v0 = (e0 - HW * m0 * m0) * (1.0 / (HW - 1))
        v1 = (e1 - HW * m1 * m1) * (1.0 / (HW - 1))
        istd_lane = jnp.where(even, jax.lax.rsqrt(v0), jax.lax.rsqrt(v1))
        xn = ((xi - mean_lane) * istd_lane).astype(jnp.bfloat16)
        planes = jnp.dot(xn, s3_ref[...], preferred_element_type=jnp.float32)
        G = min(32, NB)
        WG = G * W
        ones_row = jnp.ones((1, WG), jnp.float32)
        ones_bf = jnp.ones((1, WG), jnp.bfloat16)
        def rslice(r, k):
            lo, hi = k * W, (k + 1) * W
            if r < 0:
                return jnp.concatenate(
                    [jnp.zeros((-r, W), jnp.float32), planes[0:r + _TH, lo:hi]],
                    axis=0)
            if r + _TH > H:
                return jnp.concatenate(
                    [planes[r:H, lo:hi], jnp.zeros((r + _TH - H, W), jnp.float32)],
                    axis=0)
            return planes[r:r + _TH, lo:hi]

        y0s = []
        y1s = []
        for t in range(NB // G):
            r0 = t * G * _TH
            pieces = [
                jnp.concatenate(
                    [rslice(r0 + g * _TH + dh - 1, k) for g in range(G)], axis=1)
                for k in range(6) for dh in range(3)
            ]
            pieces.append(ones_row)
            p_s = jnp.concatenate(pieces, axis=0).astype(jnp.bfloat16)
            h1 = jnp.dot(wbd_ref[...], p_s, preferred_element_type=jnp.float32)
            hb = h1.astype(jnp.bfloat16)
            sb = hb * jax.nn.sigmoid(hb)
            h2 = jnp.concatenate([sb, ones_bf], axis=0)
            y2 = jnp.dot(w2bd_ref[...], h2, preferred_element_type=jnp.float32)
            for g in range(G):
                y0s.append(y2[0:_TH, g * W:(g + 1) * W])
                y1s.append(y2[_TH:2 * _TH, g * W:(g + 1) * W])
        y0 = jnp.concatenate(y0s, axis=0)
        y1 = jnp.concatenate(y1s, axis=0)
        zcat = jnp.concatenate([y0, y1], axis=1).astype(jnp.bfloat16)
        out = jnp.dot(zcat, r_ref[...], preferred_element_type=jnp.float32)
        std_lane = jnp.where(even, jnp.sqrt(v0), jnp.sqrt(v1))
        return out * std_lane + mean_lane

    def _body(x_ref, s3_ref, wbd_ref, w2bd_ref, r_ref, o_ref):
        for i in range(x_ref.shape[0]):
            o_ref[i] = _one_image(x_ref[i], s3_ref, wbd_ref, w2bd_ref, r_ref)

    return _body


def kernel(x, w1, b1, w2, b2):
    b, c, H, W, two = x.shape
    n = w1.shape[0]
    TH = _TH
    eye = jnp.eye(TH, dtype=jnp.float32)
    w1t = w1.reshape(n, 2, 3, 3)
    wbd = jnp.einsum("ncxw,hk->hncwxk", w1t, eye).reshape(TH * n, 144)
    bias1 = jnp.tile(b1.reshape(n), (TH,)).reshape(TH * n, 1)
    wbd = jnp.concatenate([wbd, bias1], axis=1).astype(jnp.bfloat16)
    w2bd = jnp.einsum("cn,hk->chkn", w2, eye).reshape(2 * TH, TH * n)
    bias2 = jnp.repeat(b2.reshape(2), TH).reshape(2 * TH, 1)
    w2bd = jnp.concatenate([w2bd, bias2], axis=1).astype(jnp.bfloat16)
    jj = jnp.arange(6 * W)
    c_of = jj // (3 * W)
    dw_of = (jj // W) % 3
    w_of = jj % W
    wsrc = w_of + dw_of - 1
    src = 2 * wsrc + c_of
    valid = (wsrc >= 0) & (wsrc < W)
    s3 = ((jnp.arange(2 * W)[:, None] == src[None, :]) & valid[None, :]
          ).astype(jnp.bfloat16)
    jo = jnp.arange(2 * W)
    rsrc = (jo % 2) * W + jo // 2
    rmat = (jnp.arange(2 * W)[:, None] == rsrc[None, :]).astype(jnp.bfloat16)

    xi = x.reshape(b, H, 2 * W)
    body = _make_fused_kernel(H, W, n)
    BB = 4 if b % 4 == 0 else 1

    def _run(xpart):
        nb = xpart.shape[0]
        return pl.pallas_call(
            body,
            out_shape=jax.ShapeDtypeStruct((nb, H, 2 * W), jnp.float32),
            grid=(nb // BB,),
            in_specs=[
                pl.BlockSpec((BB, H, 2 * W), lambda i: (i, 0, 0)),
                pl.BlockSpec((2 * W, 6 * W), lambda i: (0, 0)),
                pl.BlockSpec((TH * n, 145), lambda i: (0, 0)),
                pl.BlockSpec((2 * TH, TH * n + 1), lambda i: (0, 0)),
                pl.BlockSpec((2 * W, 2 * W), lambda i: (0, 0)),
            ],
            out_specs=pl.BlockSpec((BB, H, 2 * W), lambda i: (i, 0, 0)),
            compiler_params=pltpu.CompilerParams(
                dimension_semantics=("parallel",),
                vmem_limit_bytes=_VMEM_LIMIT),
        )(xpart, s3, wbd, w2bd, rmat)

    out = _run(xi)
    return out.reshape(b, c, H, W, 2)
